# baseline (speedup 1.0000x reference)
import functools

import jax
import jax.numpy as jnp
from jax import lax
from jax.experimental import pallas as pl
from jax.experimental.pallas import tpu as pltpu
from jax.experimental.pallas import tpu_sc as plsc

B = 16384
D = 64
CTX = 10
NEG = 5
NV = 1 + NEG
NC = 2
NS = 16
NW = NC * NS
PERW = B // NW
C = 32
NCHUNK = PERW // C

U_GROUPS = [(0, 128), (128, 128), (256, 64)]
V_GROUPS = [(0, 128), (128, 64)]

_mesh = plsc.VectorSubcoreMesh(
    core_axis_name="c", subcore_axis_name="s", num_cores=NC, num_subcores=NS)

_SC_PARAMS = pltpu.CompilerParams(
    needs_layout_passes=False, use_tc_tiling_on_sc=False,
    skip_device_barrier=True)


@functools.partial(
    pl.kernel,
    out_type=jax.ShapeDtypeStruct((B * D,), jnp.float32),
    mesh=_mesh,
    scratch_types=[
        pltpu.VMEM((PERW * CTX,), jnp.int32),
        pltpu.VMEM((2, C * CTX, D), jnp.float32),
        pltpu.VMEM((PERW * D,), jnp.float32),
        pltpu.SemaphoreType.DMA,
        pltpu.SemaphoreType.DMA,
    ],
    compiler_params=_SC_PARAMS,
)
def _sc_pool(uidx_hbm, u_tab, out_hbm, uidx_v, urows, uemb, sem0, sem1):
    wid = lax.axis_index("s") * NC + lax.axis_index("c")
    base = wid * PERW
    pltpu.sync_copy(uidx_hbm.at[pl.ds(base * CTX, PERW * CTX)], uidx_v)

    def issue(ch, par, sem):
        for off, n in U_GROUPS:
            pltpu.async_copy(
                u_tab.at[uidx_v.at[pl.ds(ch * (C * CTX) + off, n)]],
                urows.at[par, pl.ds(off, n)], sem)

    def drain(ch, par, sem):
        for off, n in U_GROUPS:
            pltpu.make_async_copy(
                u_tab.at[uidx_v.at[pl.ds(ch * (C * CTX) + off, n)]],
                urows.at[par, pl.ds(off, n)], sem).wait()

    def compute(ch, par):
        def elem(e, carry2):
            urow0 = e * CTX
            acc = [urows[par, urow0, pl.ds(k * 16, 16)] for k in range(4)]
            for c in range(1, CTX):
                for k in range(4):
                    acc[k] = acc[k] + urows[par, urow0 + c, pl.ds(k * 16, 16)]
            ebase = (ch * C + e) * D
            for k in range(4):
                uemb[pl.ds(ebase + k * 16, 16)] = acc[k] * (1.0 / CTX)
            return carry2

        lax.fori_loop(0, C, elem, 0)

    issue(0, 0, sem0)
    issue(1, 1, sem1)

    def pair(i, carry):
        ch0 = i * 2
        drain(ch0, 0, sem0)

        @pl.when(ch0 + 2 < NCHUNK)
        def _n0():
            issue(ch0 + 2, 0, sem0)

        compute(ch0, 0)
        drain(ch0 + 1, 1, sem1)

        @pl.when(ch0 + 3 < NCHUNK)
        def _n1():
            issue(ch0 + 3, 1, sem1)

        compute(ch0 + 1, 1)
        return carry

    lax.fori_loop(0, NCHUNK // 2, pair, 0)
    pltpu.sync_copy(uemb, out_hbm.at[pl.ds(base * D, PERW * D)])


@functools.partial(
    pl.kernel,
    out_type=jax.ShapeDtypeStruct((B * NV,), jnp.float32),
    mesh=_mesh,
    scratch_types=[
        pltpu.VMEM((PERW * NV,), jnp.int32),
        pltpu.VMEM((2, C * NV, D), jnp.float32),
        pltpu.VMEM((PERW * D,), jnp.float32),
        pltpu.VMEM((PERW * NV,), jnp.float32),
        pltpu.SemaphoreType.DMA,
        pltpu.SemaphoreType.DMA,
    ],
    compiler_params=_SC_PARAMS,
)
def _sc_dots(vidx_hbm, uemb_hbm, v_tab, out_hbm,
             vidx_v, vrows, uemb, scores, sem0, sem1):
    wid = lax.axis_index("s") * NC + lax.axis_index("c")
    base = wid * PERW
    lane0 = lax.iota(jnp.int32, 16) == 0
    pltpu.sync_copy(vidx_hbm.at[pl.ds(base * NV, PERW * NV)], vidx_v)
    pltpu.sync_copy(uemb_hbm.at[pl.ds(base * D, PERW * D)], uemb)

    def issue(ch, par, sem):
        for off, n in V_GROUPS:
            pltpu.async_copy(
                v_tab.at[vidx_v.at[pl.ds(ch * (C * NV) + off, n)]],
                vrows.at[par, pl.ds(off, n)], sem)

    def drain(ch, par, sem):
        for off, n in V_GROUPS:
            pltpu.make_async_copy(
                v_tab.at[vidx_v.at[pl.ds(ch * (C * NV) + off, n)]],
                vrows.at[par, pl.ds(off, n)], sem).wait()

    def compute(ch, par):
        def elem(e, carry2):
            ebase = (ch * C + e) * D
            acc = [uemb[pl.ds(ebase + k * 16, 16)] for k in range(4)]
            vrow0 = e * NV
            sbase = (ch * C + e) * NV
            for t in range(NV):
                prods = [vrows[par, vrow0 + t, pl.ds(k * 16, 16)] * acc[k]
                         for k in range(4)]
                s = (prods[0] + prods[1]) + (prods[2] + prods[3])
                dot = jnp.sum(s)
                dot = dot if t == 0 else -dot
                plsc.store_scatter(
                    scores,
                    [jnp.full((16,), sbase + t, dtype=jnp.int32)],
                    jnp.full((16,), dot, dtype=jnp.float32),
                    mask=lane0)
            return carry2

        lax.fori_loop(0, C, elem, 0)

    issue(0, 0, sem0)
    issue(1, 1, sem1)

    def pair(i, carry):
        ch0 = i * 2
        drain(ch0, 0, sem0)

        @pl.when(ch0 + 2 < NCHUNK)
        def _n0():
            issue(ch0 + 2, 0, sem0)

        compute(ch0, 0)
        drain(ch0 + 1, 1, sem1)

        @pl.when(ch0 + 3 < NCHUNK)
        def _n1():
            issue(ch0 + 3, 1, sem1)

        compute(ch0 + 1, 1)
        return carry

    lax.fori_loop(0, NCHUNK // 2, pair, 0)
    pltpu.sync_copy(scores, out_hbm.at[pl.ds(base * NV, PERW * NV)])


def _loss_body(x_ref, o_ref):
    o_ref[0, 0] = -jnp.sum(jax.nn.log_sigmoid(x_ref[...]))


_loss = pl.pallas_call(
    _loss_body,
    out_shape=jax.ShapeDtypeStruct((1, 1), jnp.float32),
    out_specs=pl.BlockSpec(memory_space=pltpu.SMEM),
)


def kernel(batch_0, batch_1, batch_2, u_table, v_table):
    uidx = batch_0.astype(jnp.int32).reshape(B * CTX)
    vidx = jnp.concatenate(
        [batch_1[:, None], batch_2], axis=1).astype(jnp.int32).reshape(B * NV)
    uemb = _sc_pool(uidx, u_table)
    scores = _sc_dots(vidx, uemb, v_table)
    loss = _loss(scores.reshape(B * NV // 128, 128))
    return loss.reshape(())
